# baseline (speedup 1.0000x reference)
import functools

import jax
import jax.numpy as jnp
from jax import lax
from jax.experimental import pallas as pl
from jax.experimental.pallas import tpu as pltpu
from jax.experimental.pallas import tpu_sc as plsc

B = 1024
L = 512
D = 128
NC = 2
NS = 16
NW = NC * NS
BPW = B // NW
CH = 128
CPB = L // CH
CPW = BPW * CPB


def _pos_body(table_hbm, len_hbm, out_hbm,
              lens_v, idx0, idx1, rows0, rows1,
              gsem0, gsem1, osem0, osem1):
    wid = lax.axis_index("s") * NC + lax.axis_index("c")
    pltpu.sync_copy(len_hbm.at[pl.ds(wid * BPW, BPW)], lens_v)
    out_base = wid * (BPW * L)

    lane = lax.iota(jnp.int32, 16)
    lens_lo = lens_v[pl.ds(0, 16)]
    lens_hi = lens_v[pl.ds(16, 16)]

    def build_idx(idx_ref, c):
        b_loc = c // CPB
        len_s = jnp.maximum(
            jnp.max(jnp.where(lane == b_loc, lens_lo, 0)),
            jnp.max(jnp.where(lane + 16 == b_loc, lens_hi, 0)))
        p0 = (c % CPB) * CH + 1
        for j in range(CH // 16):
            pos = lane + (p0 + j * 16)
            idx_ref[pl.ds(j * 16, 16)] = jnp.where(pos <= len_s, pos, 0)

    def body(g, carry):
        c0 = g * 2
        c1 = c0 + 1
        build_idx(idx0, c0)
        g0 = pltpu.async_copy(table_hbm.at[idx0], rows0, gsem0)
        build_idx(idx1, c1)
        g1 = pltpu.async_copy(table_hbm.at[idx1], rows1, gsem1)
        g0.wait()
        o0 = pltpu.async_copy(
            rows0, out_hbm.at[pl.ds(out_base + c0 * CH, CH)], osem0)
        g1.wait()
        o1 = pltpu.async_copy(
            rows1, out_hbm.at[pl.ds(out_base + c1 * CH, CH)], osem1)
        o0.wait()
        o1.wait()
        return carry

    lax.fori_loop(0, CPW // 2, body, 0)


@functools.partial(jax.jit)
def kernel(pos_enc, input_len):
    mesh = plsc.VectorSubcoreMesh(core_axis_name="c", subcore_axis_name="s")
    kfn = pl.kernel(
        _pos_body,
        out_type=jax.ShapeDtypeStruct((B * L, D), jnp.float32),
        mesh=mesh,
        scratch_types=[
            pltpu.VMEM((BPW,), jnp.int32),
            pltpu.VMEM((CH,), jnp.int32),
            pltpu.VMEM((CH,), jnp.int32),
            pltpu.VMEM((CH, D), jnp.float32),
            pltpu.VMEM((CH, D), jnp.float32),
            pltpu.SemaphoreType.DMA,
            pltpu.SemaphoreType.DMA,
            pltpu.SemaphoreType.DMA,
            pltpu.SemaphoreType.DMA,
        ],
        compiler_params=pltpu.CompilerParams(needs_layout_passes=False),
    )
    out = kfn(pos_enc, input_len.astype(jnp.int32))
    return out.reshape(B, L, D)


# speedup vs baseline: 99.7652x; 99.7652x over previous
import functools

import jax
import jax.numpy as jnp
from jax import lax
from jax.experimental import pallas as pl
from jax.experimental.pallas import tpu as pltpu
from jax.experimental.pallas import tpu_sc as plsc

B = 1024
L = 512
D = 128
NC = 2
NS = 16
NW = NC * NS
BPW = B // NW
ZR = 128


def _pos_body(table_hbm, len_hbm, out_hbm, lens_v, tbuf, zbuf, bbuf, sem, dsem):
    wid = lax.axis_index("s") * NC + lax.axis_index("c")
    pltpu.sync_copy(len_hbm.at[pl.ds(wid * BPW, BPW)], lens_v)
    tld = pltpu.async_copy(table_hbm, tbuf, dsem)

    zero = jnp.zeros((16,), jnp.float32)

    def zrow(r, carry):
        for j in range(D // 16):
            zbuf[r, pl.ds(j * 16, 16)] = zero
        return carry

    lax.fori_loop(0, ZR, zrow, 0)
    tld.wait()

    lane = lax.iota(jnp.int32, 16)
    lens_lo = lens_v[pl.ds(0, 16)]
    lens_hi = lens_v[pl.ds(16, 16)]
    obase = wid * (BPW * L)

    def batch_body(b_loc, carry):
        len_s = jnp.maximum(
            jnp.max(jnp.where(lane == b_loc, lens_lo, 0)),
            jnp.max(jnp.where(lane + 16 == b_loc, lens_hi, 0)))
        row0 = obase + b_loc * L
        q8 = len_s & ~7
        r8 = len_s & 7

        off = jnp.int32(0)
        for s in (256, 128, 64, 32, 16, 8):
            bit = q8 & s

            @pl.when(bit != 0)
            def _():
                pltpu.async_copy(
                    tbuf.at[pl.ds(pl.multiple_of(off, 8), s)],
                    out_hbm.at[pl.ds(pl.multiple_of(row0 + off, 8), s)], sem)

            off = off + bit

        bb = b_loc * 8
        for j in range(8):
            keep = j < r8
            for k in range(D // 16):
                tv = tbuf[q8 + j, pl.ds(k * 16, 16)]
                bbuf[bb + j, pl.ds(k * 16, 16)] = jnp.where(keep, tv, zero)
        pltpu.async_copy(
            bbuf.at[pl.ds(bb, 8)], out_hbm.at[pl.ds(pl.multiple_of(row0 + q8, 8), 8)], sem)
        off = off + 8

        t = L - q8 - 8
        for _i in range(3):
            c = t >= ZR

            @pl.when(c)
            def _():
                pltpu.async_copy(
                    zbuf, out_hbm.at[pl.ds(pl.multiple_of(row0 + off, 8), ZR)], sem)

            dec = jnp.where(c, ZR, 0).astype(jnp.int32)
            off = off + dec
            t = t - dec
        for s in (64, 32, 16, 8):
            bit = t & s

            @pl.when(bit != 0)
            def _():
                pltpu.async_copy(
                    zbuf.at[pl.ds(0, s)],
                    out_hbm.at[pl.ds(pl.multiple_of(row0 + off, 8), s)], sem)

            off = off + bit
        return carry

    lax.fori_loop(0, BPW, batch_body, 0)

    def drain(i, carry):
        pltpu.make_async_copy(table_hbm, tbuf, sem).wait()
        return carry

    lax.fori_loop(0, BPW, drain, 0)


@functools.partial(jax.jit)
def kernel(pos_enc, input_len):
    mesh = plsc.VectorSubcoreMesh(core_axis_name="c", subcore_axis_name="s")
    kfn = pl.kernel(
        _pos_body,
        out_type=jax.ShapeDtypeStruct((B * L, D), jnp.float32),
        mesh=mesh,
        scratch_types=[
            pltpu.VMEM((BPW,), jnp.int32),
            pltpu.VMEM((L, D), jnp.float32),
            pltpu.VMEM((ZR, D), jnp.float32),
            pltpu.VMEM((BPW * 8, D), jnp.float32),
            pltpu.SemaphoreType.DMA,
            pltpu.SemaphoreType.DMA,
        ],
        compiler_params=pltpu.CompilerParams(needs_layout_passes=False),
    )
    out = kfn(pos_enc[1:L + 1], input_len.astype(jnp.int32))
    return out.reshape(B, L, D)
